# baseline (speedup 1.0000x reference)
import functools

import jax
import jax.numpy as jnp
from jax import lax
from jax.experimental import pallas as pl
from jax.experimental.pallas import tpu as pltpu
from jax.experimental.pallas import tpu_sc as plsc

N_NODES = 100000
MEM_DIM = 128
MSG_DIM = 256
B = 16384

NC = 2
NS = 16
NW = NC * NS
B_PER_W = B // NW
NCHUNK = 4
CHUNK = B_PER_W // NCHUNK

_MESH = functools.partial(
    plsc.VectorSubcoreMesh, core_axis_name="c", subcore_axis_name="s"
)


def _worker_id():
  return lax.axis_index("s") * NC + lax.axis_index("c")


@functools.partial(
    pl.kernel,
    mesh=_MESH(),
    out_type=jax.ShapeDtypeStruct((B, MEM_DIM), jnp.float32),
    scratch_types=[
        pltpu.VMEM((NCHUNK, CHUNK), jnp.int32),
        pltpu.VMEM((B_PER_W, MEM_DIM), jnp.float32),
    ]
    + [pltpu.SemaphoreType.DMA] * 5,
)
def _sc_gather(mem_hbm, nids_hbm, out_hbm,
               idx_v, rows_v, s0, s1, s2, s3, ss):
  wid = _worker_id()
  base = wid * B_PER_W
  pltpu.sync_copy(nids_hbm.at[wid], idx_v)
  sems = (s0, s1, s2, s3)
  gathers = []
  for k in range(NCHUNK):
    gathers.append(
        pltpu.async_copy(
            mem_hbm.at[idx_v.at[k]],
            rows_v.at[pl.ds(k * CHUNK, CHUNK)],
            sems[k],
        )
    )
  stores = []
  for k in range(NCHUNK):
    gathers[k].wait()
    stores.append(
        pltpu.async_copy(
            rows_v.at[pl.ds(k * CHUNK, CHUNK)],
            out_hbm.at[pl.ds(base + k * CHUNK, CHUNK)],
            ss,
        )
    )
  for c in stores:
    c.wait()


_HALF = N_NODES // NC
_NPAIR = NS // 2
_PAIR_ROWS = 6256
_PAIR_LAST = _HALF - (_NPAIR - 1) * _PAIR_ROWS
_CC = 384
_SIZES = [_CC] * 16 + [_PAIR_ROWS - 16 * _CC]
_SIZES_LAST = [_CC] * 16 + [_PAIR_LAST - 16 * _CC]
_NROUND = len(_SIZES)
_OFFS = [k * _CC for k in range(_NROUND)]


@functools.partial(
    pl.kernel,
    mesh=_MESH(),
    out_type=jax.ShapeDtypeStruct((N_NODES, MEM_DIM), jnp.float32),
    scratch_types=[
        pltpu.VMEM((NCHUNK, CHUNK), jnp.int32),
        pltpu.VMEM((CHUNK,), jnp.float32),
        pltpu.VMEM_SHARED((2 * _NPAIR * _CC, MEM_DIM), jnp.float32),
        pltpu.SemaphoreType.DMA,
    ],
)
def _sc_copy_lu(mem_hbm, nids_hbm, tvals_hbm, lu_hbm, out_hbm,
                idx_v, tv_v, spmem, sl):
  c = lax.axis_index("c")
  s = lax.axis_index("s")
  wid = s * NC + c
  pltpu.sync_copy(nids_hbm.at[wid], idx_v)
  pltpu.sync_copy(tvals_hbm, tv_v)
  lu_writes = [
      pltpu.async_copy(tv_v, lu_hbm.at[idx_v.at[k]], sl)
      for k in range(NCHUNK)
  ]

  pair = s % _NPAIR
  is_prod = s < _NPAIR
  is_last = pair == _NPAIR - 1
  base = pl.multiple_of(c * _HALF + pair * _PAIR_ROWS, 8)

  def _slot(parity, size):
    off = pl.multiple_of((pair * 2 + parity) * _CC, 8)
    return spmem.at[pl.ds(off, size)]

  for r in range(_NROUND + 1):
    if r < _NROUND:
      sz, szl = _SIZES[r], _SIZES_LAST[r]

      @pl.when(jnp.logical_and(is_prod, jnp.logical_not(is_last)))
      def _(r=r, sz=sz):
        pltpu.sync_copy(mem_hbm.at[pl.ds(base + _OFFS[r], sz)],
                        _slot(r % 2, sz))

      @pl.when(jnp.logical_and(is_prod, is_last))
      def _(r=r, szl=szl):
        pltpu.sync_copy(mem_hbm.at[pl.ds(base + _OFFS[r], szl)],
                        _slot(r % 2, szl))

    if r >= 1:
      sz, szl = _SIZES[r - 1], _SIZES_LAST[r - 1]

      @pl.when(jnp.logical_and(jnp.logical_not(is_prod),
                               jnp.logical_not(is_last)))
      def _(r=r, sz=sz):
        pltpu.sync_copy(_slot((r - 1) % 2, sz),
                        out_hbm.at[pl.ds(base + _OFFS[r - 1], sz)])

      @pl.when(jnp.logical_and(jnp.logical_not(is_prod), is_last))
      def _(r=r, szl=szl):
        pltpu.sync_copy(_slot((r - 1) % 2, szl),
                        out_hbm.at[pl.ds(base + _OFFS[r - 1], szl)])

    plsc.subcore_barrier()

  for cp in lu_writes:
    cp.wait()


_BM = 1024
_GRID = B // _BM


def _gru_body(msg_ref, h_ref, wi_ref, wh_ref, bi_ref, bh_ref, out_ref):
  gi = (
      jnp.dot(msg_ref[...], wi_ref[...], preferred_element_type=jnp.float32)
      + bi_ref[...]
  )
  gh = (
      jnp.dot(h_ref[...], wh_ref[...], preferred_element_type=jnp.float32)
      + bh_ref[...]
  )
  H = MEM_DIM
  r = jax.nn.sigmoid(gi[:, :H] + gh[:, :H])
  z = jax.nn.sigmoid(gi[:, H : 2 * H] + gh[:, H : 2 * H])
  n = jnp.tanh(gi[:, 2 * H :] + r * gh[:, 2 * H :])
  out_ref[...] = (1.0 - z) * n + z * h_ref[...]


def _tc_gru(msg, h, wi_t, wh_t, bi, bh):
  return pl.pallas_call(
      _gru_body,
      grid=(_GRID,),
      in_specs=[
          pl.BlockSpec((_BM, MSG_DIM), lambda i: (i, 0)),
          pl.BlockSpec((_BM, MEM_DIM), lambda i: (i, 0)),
          pl.BlockSpec((MSG_DIM, 3 * MEM_DIM), lambda i: (0, 0)),
          pl.BlockSpec((MEM_DIM, 3 * MEM_DIM), lambda i: (0, 0)),
          pl.BlockSpec((1, 3 * MEM_DIM), lambda i: (0, 0)),
          pl.BlockSpec((1, 3 * MEM_DIM), lambda i: (0, 0)),
      ],
      out_specs=pl.BlockSpec((_BM, MEM_DIM), lambda i: (i, 0)),
      out_shape=jax.ShapeDtypeStruct((B, MEM_DIM), jnp.float32),
  )(msg, h, wi_t, wh_t, bi, bh)


@functools.partial(
    pl.kernel,
    mesh=_MESH(),
    out_type=(),
    scratch_types=[
        pltpu.VMEM((NCHUNK, CHUNK), jnp.int32),
        pltpu.VMEM((B_PER_W, MEM_DIM), jnp.float32),
    ]
    + [pltpu.SemaphoreType.DMA] * 5,
)
def _sc_scatter(nids_hbm, hnew_hbm, mem_hbm,
                idx_v, rows_v, s0, s1, s2, s3, ss):
  wid = _worker_id()
  base = wid * B_PER_W
  pltpu.sync_copy(nids_hbm.at[wid], idx_v)
  sems = (s0, s1, s2, s3)
  loads = []
  for k in range(NCHUNK):
    loads.append(
        pltpu.async_copy(
            hnew_hbm.at[pl.ds(base + k * CHUNK, CHUNK)],
            rows_v.at[pl.ds(k * CHUNK, CHUNK)],
            sems[k],
        )
    )
  scatters = []
  for k in range(NCHUNK):
    loads[k].wait()
    scatters.append(
        pltpu.async_copy(
            rows_v.at[pl.ds(k * CHUNK, CHUNK)],
            mem_hbm.at[idx_v.at[k]],
            ss,
        )
    )
  for c in scatters:
    c.wait()


def kernel(unique_nids, unique_msg, time, memory, last_update,
           W_ih, W_hh, b_ih, b_hh):
  nids3 = unique_nids.astype(jnp.int32).reshape(NW, NCHUNK, CHUNK)
  tvals = jnp.full((CHUNK,), time, dtype=jnp.float32)
  lu_ref = jax.new_ref(last_update)
  h = _sc_gather(memory, nids3)
  h_new = _tc_gru(
      unique_msg, h, W_ih.T, W_hh.T,
      b_ih.reshape(1, -1), b_hh.reshape(1, -1),
  )
  mem_copy = _sc_copy_lu(memory, nids3, tvals, lu_ref)
  mem_ref = jax.new_ref(mem_copy)
  _sc_scatter(nids3, h_new, mem_ref)
  return jax.freeze(mem_ref), jax.freeze(lu_ref)
